# Initial kernel scaffold; baseline (speedup 1.0000x reference)
import jax
import jax.numpy as jnp
from jax.experimental import pallas as pl


def kernel(hidden_states, weight):
    raise NotImplementedError("write your pallas kernel here")


# speedup vs baseline: 1.3932x; 1.3932x over previous
import functools

import jax
import jax.numpy as jnp
from jax.experimental import pallas as pl

TOPK = 2
EPS = 1e-20


def _router_kernel(x_ref, wt_ref, idx_ref, w_ref):
    x = x_ref[...]
    wt = wt_ref[...]
    logits = jnp.dot(x, wt, preferred_element_type=jnp.float32)

    m = jnp.max(logits, axis=-1, keepdims=True)
    e = jnp.exp(logits - m)
    s = e / jnp.sum(e, axis=-1, keepdims=True)

    ncols = s.shape[-1]
    col = jax.lax.broadcasted_iota(jnp.int32, s.shape, dimension=1)

    i1 = jnp.argmax(s, axis=-1).astype(jnp.int32)
    v1 = jnp.max(s, axis=-1)
    masked = jnp.where(col == i1[:, None], -jnp.inf, s)
    i2 = jnp.argmax(masked, axis=-1).astype(jnp.int32)
    v2 = jnp.max(masked, axis=-1)

    denom = v1 + v2 + EPS
    idx_ref[...] = jnp.stack([i1, i2], axis=-1)
    w_ref[...] = jnp.stack([v1 / denom, v2 / denom], axis=-1)
    del ncols


@functools.partial(jax.jit, static_argnames=("tile",))
def _route(x2d, wt, tile):
    S, H = x2d.shape
    E = wt.shape[1]
    grid = (S // tile,)
    return pl.pallas_call(
        _router_kernel,
        grid=grid,
        in_specs=[
            pl.BlockSpec((tile, H), lambda i: (i, 0)),
            pl.BlockSpec((H, E), lambda i: (0, 0)),
        ],
        out_specs=[
            pl.BlockSpec((tile, TOPK), lambda i: (i, 0)),
            pl.BlockSpec((tile, TOPK), lambda i: (i, 0)),
        ],
        out_shape=[
            jax.ShapeDtypeStruct((S, TOPK), jnp.int32),
            jax.ShapeDtypeStruct((S, TOPK), jnp.float32),
        ],
    )(x2d, wt)


def kernel(hidden_states, weight):
    bsz, seq_len, h = hidden_states.shape
    x2d = hidden_states.reshape(-1, h).astype(jnp.float32)
    wt = weight.astype(jnp.float32).T
    topk_idx, topk_weight = _route(x2d, wt, tile=512)
    return (topk_idx, topk_weight)


# speedup vs baseline: 1.4111x; 1.0129x over previous
import functools

import jax
import jax.numpy as jnp
from jax.experimental import pallas as pl

TOPK = 2
EPS = 1e-20


def _router_kernel(x_ref, wt_ref, idx_ref, w_ref):
    x = x_ref[...]
    wt = wt_ref[...]
    logits = jnp.dot(x, wt, preferred_element_type=jnp.float32)

    m = jnp.max(logits, axis=-1, keepdims=True)
    e = jnp.exp(logits - m)
    s = e / jnp.sum(e, axis=-1, keepdims=True)

    ncols = s.shape[-1]
    col = jax.lax.broadcasted_iota(jnp.int32, s.shape, dimension=1)

    i1 = jnp.argmax(s, axis=-1).astype(jnp.int32)
    v1 = jnp.max(s, axis=-1)
    masked = jnp.where(col == i1[:, None], -jnp.inf, s)
    i2 = jnp.argmax(masked, axis=-1).astype(jnp.int32)
    v2 = jnp.max(masked, axis=-1)

    denom = v1 + v2 + EPS
    idx_ref[...] = jnp.stack([i1, i2], axis=-1)
    w_ref[...] = jnp.stack([v1 / denom, v2 / denom], axis=-1)
    del ncols


@functools.partial(jax.jit, static_argnames=("tile",))
def _route(x2d, wt, tile):
    S, H = x2d.shape
    E = wt.shape[1]
    grid = (S // tile,)
    return pl.pallas_call(
        _router_kernel,
        grid=grid,
        in_specs=[
            pl.BlockSpec((tile, H), lambda i: (i, 0)),
            pl.BlockSpec((H, E), lambda i: (0, 0)),
        ],
        out_specs=[
            pl.BlockSpec((tile, TOPK), lambda i: (i, 0)),
            pl.BlockSpec((tile, TOPK), lambda i: (i, 0)),
        ],
        out_shape=[
            jax.ShapeDtypeStruct((S, TOPK), jnp.int32),
            jax.ShapeDtypeStruct((S, TOPK), jnp.float32),
        ],
    )(x2d, wt)


def kernel(hidden_states, weight):
    bsz, seq_len, h = hidden_states.shape
    x2d = hidden_states.reshape(-1, h).astype(jnp.float32)
    wt = weight.astype(jnp.float32).T
    topk_idx, topk_weight = _route(x2d, wt, tile=1024)
    return (topk_idx, topk_weight)


# speedup vs baseline: 1.5018x; 1.0643x over previous
import functools

import jax
import jax.numpy as jnp
from jax.experimental import pallas as pl
from jax.experimental.pallas import tpu as pltpu

TOPK = 2
EPS = 1e-20


def _router_kernel(x_ref, w_ref_in, idx_ref, w_ref):
    x = x_ref[...]
    w = w_ref_in[...]
    logits = jax.lax.dot_general(
        x, w, (((1,), (1,)), ((), ())),
        preferred_element_type=jnp.float32)

    m = jnp.max(logits, axis=-1, keepdims=True)
    e = jnp.exp(logits - m)
    s = e / jnp.sum(e, axis=-1, keepdims=True)

    col = jax.lax.broadcasted_iota(jnp.int32, s.shape, dimension=1)

    i1 = jnp.argmax(s, axis=-1).astype(jnp.int32)
    v1 = jnp.max(s, axis=-1)
    masked = jnp.where(col == i1[:, None], -jnp.inf, s)
    i2 = jnp.argmax(masked, axis=-1).astype(jnp.int32)
    v2 = jnp.max(masked, axis=-1)

    denom = v1 + v2 + EPS
    idx_ref[...] = jnp.stack([i1, i2], axis=-1)
    w_ref[...] = jnp.stack([v1 / denom, v2 / denom], axis=-1)


@functools.partial(jax.jit, static_argnames=("tile",))
def _route(x2d, w, tile):
    S, H = x2d.shape
    E = w.shape[0]
    grid = (S // tile,)
    return pl.pallas_call(
        _router_kernel,
        grid=grid,
        in_specs=[
            pl.BlockSpec((tile, H), lambda i: (i, 0)),
            pl.BlockSpec((E, H), lambda i: (0, 0)),
        ],
        out_specs=[
            pl.BlockSpec((tile, TOPK), lambda i: (i, 0)),
            pl.BlockSpec((tile, TOPK), lambda i: (i, 0)),
        ],
        out_shape=[
            jax.ShapeDtypeStruct((S, TOPK), jnp.int32),
            jax.ShapeDtypeStruct((S, TOPK), jnp.float32),
        ],
        compiler_params=pltpu.CompilerParams(
            dimension_semantics=("parallel",)),
    )(x2d, w)


def kernel(hidden_states, weight):
    bsz, seq_len, h = hidden_states.shape
    x2d = hidden_states.reshape(-1, h).astype(jnp.float32)
    topk_idx, topk_weight = _route(x2d, weight.astype(jnp.float32), tile=1024)
    return (topk_idx, topk_weight)
